# baseline (speedup 1.0000x reference)
import functools

import numpy as np
import jax
import jax.numpy as jnp
from jax import lax
from jax.experimental import pallas as pl
from jax.experimental.pallas import tpu as pltpu
from jax.experimental.pallas import tpu_sc as plsc

_FIELD_DIMS = [100000] * 26
_F = len(_FIELD_DIMS)
_B = 16384
_NROWS = int(np.sum(_FIELD_DIMS))
_NC, _NS = 2, 16
_NW = _NC * _NS
_RPW = _B // _NW
_EPW = _RPW * _F
_GV = _RPW // 16

_OFFS = [int(v) for v in
         np.concatenate(([0], np.cumsum(_FIELD_DIMS)[:-1])).astype(np.int32)]

_CHUNKS = [(0, 2), (2, 4), (4, 15), (15, 26)]
_CHUNK_ROW0 = [_OFFS[f0] for f0, _ in _CHUNKS]


def _chunk_of(f):
    for ci, (f0, f1) in enumerate(_CHUNKS):
        if f0 <= f < f1:
            return ci
    raise AssertionError


_OFFS_LOCAL = [_OFFS[f] - _CHUNK_ROW0[_chunk_of(f)] for f in range(_F)]

_mesh = plsc.VectorSubcoreMesh(core_axis_name="c", subcore_axis_name="s")


@functools.partial(
    pl.kernel,
    out_type=jax.ShapeDtypeStruct((_NW, _EPW), jnp.int32),
    mesh=_mesh,
    compiler_params=pltpu.CompilerParams(needs_layout_passes=False),
    scratch_types=[
        pltpu.VMEM((_F, _RPW), jnp.int32),
        pltpu.VMEM((_EPW,), jnp.int32),
    ],
)
def _build_idx(xt_hbm, idx_hbm, x_v, idx_v):
    wid = lax.axis_index("s") * _NC + lax.axis_index("c")
    base = wid * _RPW

    pltpu.sync_copy(xt_hbm.at[:, pl.ds(base, _RPW)], x_v)

    def _add_offs(j, carry):
        s = pl.ds(j * 16, 16)
        for f in range(_F):
            idx_v[pl.ds(f * _RPW + j * 16, 16)] = x_v[f, s] + _OFFS_LOCAL[f]
        return carry
    lax.fori_loop(0, _GV, _add_offs, 0)

    pltpu.sync_copy(idx_v, idx_hbm.at[wid])


def _make_k2(f0, f1, first, last):
    nf = f1 - f0
    epw = nf * _RPW

    scratch = [
        pltpu.VMEM((epw,), jnp.int32),
        pltpu.VMEM((epw,), jnp.float32),
        pltpu.VMEM((_RPW,), jnp.float32),
        pltpu.SemaphoreType.DMA,
    ]
    if last:
        scratch.insert(3, pltpu.VMEM((16,), jnp.float32))

    @functools.partial(
        pl.kernel,
        out_type=jax.ShapeDtypeStruct((_B,), jnp.float32),
        mesh=_mesh,
        compiler_params=pltpu.CompilerParams(needs_layout_passes=False),
        scratch_types=scratch,
    )
    def _k2(*args):
        if first and last:
            idx_hbm, tbl_hbm, bias_hbm, out_hbm = args[:4]
            rest = args[4:]
        elif first:
            idx_hbm, tbl_hbm, out_hbm = args[:3]
            rest = args[3:]
        elif last:
            idx_hbm, tbl_hbm, acc_hbm, bias_hbm, out_hbm = args[:5]
            rest = args[5:]
        else:
            idx_hbm, tbl_hbm, acc_hbm, out_hbm = args[:4]
            rest = args[4:]
        if last:
            idx_v, val_v, acc_v, bias_v, sem = rest
        else:
            idx_v, val_v, acc_v, sem = rest

        wid = lax.axis_index("s") * _NC + lax.axis_index("c")
        base = wid * _RPW

        pltpu.sync_copy(idx_hbm.at[wid, pl.ds(f0 * _RPW, epw)], idx_v)
        if not first:
            pltpu.sync_copy(acc_hbm.at[pl.ds(base, _RPW)], acc_v)
        if last:
            pltpu.sync_copy(bias_hbm, bias_v)

        pltpu.async_copy(tbl_hbm.at[idx_v], val_v, sem).wait()

        if last:
            bias16 = bias_v[...]

        def _reduce(g, carry):
            s = pl.ds(g * 16, 16)
            if first:
                acc = jnp.zeros((16,), jnp.float32)
            else:
                acc = acc_v[s]
            for f in range(nf):
                acc = acc + val_v[pl.ds(f * _RPW + g * 16, 16)]
            if last:
                z = acc + bias16
                acc_v[s] = 1.0 / (1.0 + jnp.exp(-z))
            else:
                acc_v[s] = acc
            return carry
        lax.fori_loop(0, _GV, _reduce, 0)

        pltpu.sync_copy(acc_v, out_hbm.at[pl.ds(base, _RPW)])

    return _k2


_K2S = [
    _make_k2(f0, f1, ci == 0, ci == len(_CHUNKS) - 1)
    for ci, (f0, f1) in enumerate(_CHUNKS)
]


def kernel(x, table, bias):
    xt = x.T
    bias16 = jnp.broadcast_to(bias.astype(jnp.float32), (16,))
    idx = _build_idx(xt)
    acc = None
    n = len(_CHUNKS)
    for ci, (f0, f1) in enumerate(_CHUNKS):
        r0 = _OFFS[f0]
        r1 = _OFFS[f1] if f1 < _F else _NROWS
        tbl_c = lax.slice_in_dim(table, r0, r1, axis=0).reshape(r1 - r0)
        args = [idx, tbl_c]
        if ci > 0:
            args.append(acc)
        if ci == n - 1:
            args.append(bias16)
        acc = _K2S[ci](*args)
    return acc


# speedup vs baseline: 1.0216x; 1.0216x over previous
import functools

import numpy as np
import jax
import jax.numpy as jnp
from jax import lax
from jax.experimental import pallas as pl
from jax.experimental.pallas import tpu as pltpu
from jax.experimental.pallas import tpu_sc as plsc

_FIELD_DIMS = [100000] * 26
_F = len(_FIELD_DIMS)
_B = 16384
_NROWS = int(np.sum(_FIELD_DIMS))
_NC, _NS = 2, 16
_NW = _NC * _NS
_RPW = _B // _NW
_EPW = _RPW * _F
_GV = _RPW // 16

_OFFS = [int(v) for v in
         np.concatenate(([0], np.cumsum(_FIELD_DIMS)[:-1])).astype(np.int32)]

_CHUNKS = [(0, 5), (5, 10), (10, 18), (18, 26)]
_CHUNK_ROW0 = [_OFFS[f0] for f0, _ in _CHUNKS]


def _chunk_of(f):
    for ci, (f0, f1) in enumerate(_CHUNKS):
        if f0 <= f < f1:
            return ci
    raise AssertionError


_OFFS_LOCAL = [_OFFS[f] - _CHUNK_ROW0[_chunk_of(f)] for f in range(_F)]

_mesh = plsc.VectorSubcoreMesh(core_axis_name="c", subcore_axis_name="s")


@functools.partial(
    pl.kernel,
    out_type=jax.ShapeDtypeStruct((_NW, _EPW), jnp.int32),
    mesh=_mesh,
    compiler_params=pltpu.CompilerParams(needs_layout_passes=False),
    scratch_types=[
        pltpu.VMEM((_F, _RPW), jnp.int32),
        pltpu.VMEM((_EPW,), jnp.int32),
    ],
)
def _build_idx(xt_hbm, idx_hbm, x_v, idx_v):
    wid = lax.axis_index("s") * _NC + lax.axis_index("c")
    base = wid * _RPW

    pltpu.sync_copy(xt_hbm.at[:, pl.ds(base, _RPW)], x_v)

    def _add_offs(j, carry):
        s = pl.ds(j * 16, 16)
        for f in range(_F):
            idx_v[pl.ds(f * _RPW + j * 16, 16)] = x_v[f, s] + _OFFS_LOCAL[f]
        return carry
    lax.fori_loop(0, _GV, _add_offs, 0)

    pltpu.sync_copy(idx_v, idx_hbm.at[wid])


def _make_k2(f0, f1, first, last):
    nf = f1 - f0
    epw = nf * _RPW

    scratch = [
        pltpu.VMEM((epw,), jnp.int32),
        pltpu.VMEM((epw,), jnp.float32),
        pltpu.VMEM((_RPW,), jnp.float32),
        pltpu.SemaphoreType.DMA,
    ]
    if last:
        scratch.insert(3, pltpu.VMEM((16,), jnp.float32))

    @functools.partial(
        pl.kernel,
        out_type=jax.ShapeDtypeStruct((_B,), jnp.float32),
        mesh=_mesh,
        compiler_params=pltpu.CompilerParams(needs_layout_passes=False),
        scratch_types=scratch,
    )
    def _k2(*args):
        if first and last:
            idx_hbm, tbl_hbm, bias_hbm, out_hbm = args[:4]
            rest = args[4:]
        elif first:
            idx_hbm, tbl_hbm, out_hbm = args[:3]
            rest = args[3:]
        elif last:
            idx_hbm, tbl_hbm, acc_hbm, bias_hbm, out_hbm = args[:5]
            rest = args[5:]
        else:
            idx_hbm, tbl_hbm, acc_hbm, out_hbm = args[:4]
            rest = args[4:]
        if last:
            idx_v, val_v, acc_v, bias_v, sem = rest
        else:
            idx_v, val_v, acc_v, sem = rest

        wid = lax.axis_index("s") * _NC + lax.axis_index("c")
        base = wid * _RPW

        pltpu.sync_copy(idx_hbm.at[wid, pl.ds(f0 * _RPW, epw)], idx_v)
        if not first:
            pltpu.sync_copy(acc_hbm.at[pl.ds(base, _RPW)], acc_v)
        if last:
            pltpu.sync_copy(bias_hbm, bias_v)

        pltpu.async_copy(tbl_hbm.at[idx_v], val_v, sem).wait()

        if last:
            bias16 = bias_v[...]

        def _reduce(g, carry):
            s = pl.ds(g * 16, 16)
            if first:
                acc = jnp.zeros((16,), jnp.float32)
            else:
                acc = acc_v[s]
            for f in range(nf):
                acc = acc + val_v[pl.ds(f * _RPW + g * 16, 16)]
            if last:
                z = acc + bias16
                acc_v[s] = 1.0 / (1.0 + jnp.exp(-z))
            else:
                acc_v[s] = acc
            return carry
        lax.fori_loop(0, _GV, _reduce, 0)

        pltpu.sync_copy(acc_v, out_hbm.at[pl.ds(base, _RPW)])

    return _k2


_K2S = [
    _make_k2(f0, f1, ci == 0, ci == len(_CHUNKS) - 1)
    for ci, (f0, f1) in enumerate(_CHUNKS)
]


def kernel(x, table, bias):
    xt = x.T
    bias16 = jnp.broadcast_to(bias.astype(jnp.float32), (16,))
    idx = _build_idx(xt)
    acc = None
    n = len(_CHUNKS)
    for ci, (f0, f1) in enumerate(_CHUNKS):
        r0 = _OFFS[f0]
        r1 = _OFFS[f1] if f1 < _F else _NROWS
        tbl_c = lax.slice_in_dim(table, r0, r1, axis=0).reshape(r1 - r0)
        args = [idx, tbl_c]
        if ci > 0:
            args.append(acc)
        if ci == n - 1:
            args.append(bias16)
        acc = _K2S[ci](*args)
    return acc


# speedup vs baseline: 1.2310x; 1.2050x over previous
import functools

import numpy as np
import jax
import jax.numpy as jnp
from jax import lax
from jax.experimental import pallas as pl
from jax.experimental.pallas import tpu as pltpu
from jax.experimental.pallas import tpu_sc as plsc

_FIELD_DIMS = [100000] * 26
_F = len(_FIELD_DIMS)
_B = 16384
_NROWS = int(np.sum(_FIELD_DIMS))
_NC, _NS = 2, 16
_NW = _NC * _NS
_RPW = _B // _NW
_EPW = _RPW * _F
_GV = _RPW // 16

_OFFS = [int(v) for v in
         np.concatenate(([0], np.cumsum(_FIELD_DIMS)[:-1])).astype(np.int32)]

_CHUNKS = [(0, 6), (6, 12), (12, 19), (19, 26)]
_CHUNK_ROW0 = [_OFFS[f0] for f0, _ in _CHUNKS]


def _chunk_of(f):
    for ci, (f0, f1) in enumerate(_CHUNKS):
        if f0 <= f < f1:
            return ci
    raise AssertionError


_OFFS_LOCAL = [_OFFS[f] - _CHUNK_ROW0[_chunk_of(f)] for f in range(_F)]

_mesh = plsc.VectorSubcoreMesh(core_axis_name="c", subcore_axis_name="s")


@functools.partial(
    pl.kernel,
    out_type=jax.ShapeDtypeStruct((_NW, _EPW), jnp.int32),
    mesh=_mesh,
    compiler_params=pltpu.CompilerParams(needs_layout_passes=False),
    scratch_types=[
        pltpu.VMEM((_F, _RPW), jnp.int32),
        pltpu.VMEM((_EPW,), jnp.int32),
    ],
)
def _build_idx(xt_hbm, idx_hbm, x_v, idx_v):
    wid = lax.axis_index("s") * _NC + lax.axis_index("c")
    base = wid * _RPW

    pltpu.sync_copy(xt_hbm.at[:, pl.ds(base, _RPW)], x_v)

    def _add_offs(j, carry):
        s = pl.ds(j * 16, 16)
        for f in range(_F):
            idx_v[pl.ds(f * _RPW + j * 16, 16)] = x_v[f, s] + _OFFS_LOCAL[f]
        return carry
    lax.fori_loop(0, _GV, _add_offs, 0)

    pltpu.sync_copy(idx_v, idx_hbm.at[wid])


def _make_k2(f0, f1, first, last):
    nf = f1 - f0
    epw = nf * _RPW

    scratch = [
        pltpu.VMEM((epw,), jnp.int32),
        pltpu.VMEM((epw,), jnp.float32),
        pltpu.VMEM((_RPW,), jnp.float32),
        pltpu.SemaphoreType.DMA,
    ]
    if last:
        scratch.insert(3, pltpu.VMEM((16,), jnp.float32))

    @functools.partial(
        pl.kernel,
        out_type=jax.ShapeDtypeStruct((_B,), jnp.float32),
        mesh=_mesh,
        compiler_params=pltpu.CompilerParams(needs_layout_passes=False),
        scratch_types=scratch,
    )
    def _k2(*args):
        if first and last:
            idx_hbm, tbl_hbm, bias_hbm, out_hbm = args[:4]
            rest = args[4:]
        elif first:
            idx_hbm, tbl_hbm, out_hbm = args[:3]
            rest = args[3:]
        elif last:
            idx_hbm, tbl_hbm, acc_hbm, bias_hbm, out_hbm = args[:5]
            rest = args[5:]
        else:
            idx_hbm, tbl_hbm, acc_hbm, out_hbm = args[:4]
            rest = args[4:]
        if last:
            idx_v, val_v, acc_v, bias_v, sem = rest
        else:
            idx_v, val_v, acc_v, sem = rest

        wid = lax.axis_index("s") * _NC + lax.axis_index("c")
        base = wid * _RPW

        pltpu.sync_copy(idx_hbm.at[wid, pl.ds(f0 * _RPW, epw)], idx_v)
        if not first:
            pltpu.sync_copy(acc_hbm.at[pl.ds(base, _RPW)], acc_v)
        if last:
            pltpu.sync_copy(bias_hbm, bias_v)

        pltpu.async_copy(tbl_hbm.at[idx_v], val_v, sem).wait()

        if last:
            bias16 = bias_v[...]

        def _reduce(g, carry):
            s = pl.ds(g * 16, 16)
            if first:
                acc = jnp.zeros((16,), jnp.float32)
            else:
                acc = acc_v[s]
            for f in range(nf):
                acc = acc + val_v[pl.ds(f * _RPW + g * 16, 16)]
            if last:
                z = acc + bias16
                acc_v[s] = 1.0 / (1.0 + jnp.exp(-z))
            else:
                acc_v[s] = acc
            return carry
        lax.fori_loop(0, _GV, _reduce, 0)

        pltpu.sync_copy(acc_v, out_hbm.at[pl.ds(base, _RPW)])

    return _k2


_K2S = [
    _make_k2(f0, f1, ci == 0, ci == len(_CHUNKS) - 1)
    for ci, (f0, f1) in enumerate(_CHUNKS)
]


def kernel(x, table, bias):
    xt = x.T
    bias16 = jnp.broadcast_to(bias.astype(jnp.float32), (16,))
    idx = _build_idx(xt)
    acc = None
    n = len(_CHUNKS)
    for ci, (f0, f1) in enumerate(_CHUNKS):
        r0 = _OFFS[f0]
        r1 = _OFFS[f1] if f1 < _F else _NROWS
        tbl_c = lax.slice_in_dim(table, r0, r1, axis=0).reshape(r1 - r0)
        args = [idx, tbl_c]
        if ci > 0:
            args.append(acc)
        if ci == n - 1:
            args.append(bias16)
        acc = _K2S[ci](*args)
    return acc
